# baseline (speedup 1.0000x reference)
import functools

import jax
import jax.numpy as jnp
from jax import lax
from jax.experimental import pallas as pl
from jax.experimental.pallas import tpu as pltpu
from jax.experimental.pallas import tpu_sc as plsc

_N = 10000
_E = 320000
_D = 128
_G = 64
_C = 32

_NW = 32
_K = 80
_EPW = _E // _NW
_NCH = _EPW // _K
_DEPTH = 4
_GLAG = 2
_RPS = 624
_TAIL = _N - 16 * _RPS

_mesh = plsc.VectorSubcoreMesh(core_axis_name="c", subcore_axis_name="s")


def _sliced_copy(s, src, dst):
    pltpu.sync_copy(src.at[pl.ds(s * _RPS, _RPS)], dst.at[pl.ds(s * _RPS, _RPS)])

    @pl.when(s == 15)
    def _():
        pltpu.sync_copy(src.at[pl.ds(16 * _RPS, _TAIL)],
                        dst.at[pl.ds(16 * _RPS, _TAIL)])


@functools.partial(
    pl.kernel,
    out_type=jax.ShapeDtypeStruct((_NW, _N), jnp.float32),
    mesh=_mesh,
    compiler_params=pltpu.CompilerParams(needs_layout_passes=False),
    scratch_types=[
        pltpu.VMEM((_N,), jnp.int32),
        pltpu.VMEM((_N,), jnp.float32),
    ],
)
def _deg_sc(dst_hbm, zeros_hbm, out_hbm, idxv, hist):
    c = lax.axis_index("c")
    s = lax.axis_index("s")
    wid = s * 2 + c
    pltpu.sync_copy(dst_hbm.at[wid], idxv)
    pltpu.sync_copy(zeros_hbm, hist)
    ones = jnp.ones((16,), jnp.float32)

    def body(v, carry):
        idx = idxv[pl.ds(v * 16, 16)]
        plsc.addupdate_scatter(hist, [idx], ones)
        return carry

    lax.fori_loop(0, _EPW // 16, body, 0)
    pltpu.sync_copy(hist, out_hbm.at[wid])


_NB = 5
_BCH = _NCH // _NB


@functools.partial(
    pl.kernel,
    out_type=jax.ShapeDtypeStruct((2, _N, _D), jnp.float32),
    mesh=_mesh,
    scratch_types=[
        pltpu.VMEM((_BCH, _K), jnp.int32),
        pltpu.VMEM((_BCH, _K), jnp.int32),
        [pltpu.VMEM((_K, _D), jnp.float32)] * _DEPTH,
        pltpu.VMEM_SHARED((_N, _D), jnp.float32),
        [pltpu.SemaphoreType.DMA] * _DEPTH,
        [pltpu.SemaphoreType.DMA] * _DEPTH,
    ],
)
def _edge_sc(y_hbm, src_hbm, dst_hbm, zeros_hbm, out_hbm,
             srcv, dstv, rows, acc, gsem, ssem):
    c = lax.axis_index("c")
    s = lax.axis_index("s")
    wid = s * 2 + c
    _sliced_copy(s, zeros_hbm, acc)
    plsc.subcore_barrier()

    def gdesc(j, b):
        return pltpu.make_async_copy(y_hbm.at[srcv.at[j]], rows[b], gsem[b])

    def sdesc(j, b):
        return pltpu.make_async_copy(rows[b], acc.at[dstv.at[j]], ssem[b])

    def step(j, b):
        jmg = jnp.maximum(j - _GLAG, 0)
        jmd = jnp.maximum(j - _DEPTH, 0)

        @pl.when(jnp.logical_and(j >= _DEPTH, j <= _BCH + _DEPTH - 1))
        def _():
            sdesc(jmd, b).wait()

        @pl.when(j <= _BCH - 1)
        def _():
            gdesc(j, b).start()

        @pl.when(jnp.logical_and(j >= _GLAG, j <= _BCH + _GLAG - 1))
        def _():
            bb = (b - _GLAG) % _DEPTH
            gdesc(jmg, bb).wait()
            sdesc(jmg, bb).start(add=True)

    def body(u, carry):
        for t in range(_DEPTH):
            step(_DEPTH * u + t, t)
        return carry

    for blk in range(_NB):
        pltpu.sync_copy(src_hbm.at[wid, blk], srcv)
        pltpu.sync_copy(dst_hbm.at[wid, blk], dstv)
        lax.fori_loop(0, (_BCH + 2 * _DEPTH - 1) // _DEPTH, body, 0)

    plsc.subcore_barrier()
    _sliced_copy(s, acc, out_hbm.at[c])


def _tc1_body(x_ref, w1_ref, degp_ref, y1_ref, dinv_ref):
    deg_col = lax.dot_general(degp_ref[...], jnp.ones((_NW, 1), jnp.float32),
                              (((0,), (0,)), ((), ())),
                              preferred_element_type=jnp.float32)
    dinv = lax.rsqrt(deg_col + 1.0)
    xw = jnp.dot(x_ref[...], w1_ref[...], preferred_element_type=jnp.float32)
    y1_ref[...] = xw * dinv
    dinv_ref[...] = jnp.broadcast_to(dinv, (_N, _D))


_tc1 = pl.pallas_call(
    _tc1_body,
    out_shape=[
        jax.ShapeDtypeStruct((_N, _D), jnp.float32),
        jax.ShapeDtypeStruct((_N, _D), jnp.float32),
    ],
)


def _tc2_body(zp_ref, y1_ref, dinv_ref, b1_ref, w2_ref, y2_ref):
    z = zp_ref[0] + zp_ref[1] + y1_ref[...]
    h = jnp.maximum(z * dinv_ref[...] + b1_ref[...], 0.0)
    y2_ref[...] = jnp.dot(h, w2_ref[...],
                          preferred_element_type=jnp.float32) * dinv_ref[...]


_tc2 = pl.pallas_call(
    _tc2_body,
    out_shape=jax.ShapeDtypeStruct((_N, _D), jnp.float32),
)


def _tc3_body(zp_ref, y2_ref, dinv_ref, b2_ref, batch_ref, wct_ref, bc_ref,
              out_ref):
    z = zp_ref[0] + zp_ref[1] + y2_ref[...]
    h = jnp.maximum(z * dinv_ref[...] + b2_ref[...], 0.0)
    seg = lax.broadcasted_iota(jnp.int32, (_G, _N), 0)
    oht = (seg == batch_ref[...]).astype(jnp.float32)
    sums = jnp.dot(oht, h, preferred_element_type=jnp.float32)
    cnt = jnp.sum(oht, axis=1, keepdims=True)
    pooled = sums / jnp.maximum(cnt, 1.0)
    out_ref[...] = jnp.dot(pooled, wct_ref[...],
                           preferred_element_type=jnp.float32) + bc_ref[...]


_tc3 = pl.pallas_call(
    _tc3_body,
    out_shape=jax.ShapeDtypeStruct((_G, _C), jnp.float32),
)


def kernel(x, edge_index, batch, W1, b1, W2, b2, Wc, bc):
    src = edge_index[0].reshape(_NW, _NB, _BCH, _K)
    dst = edge_index[1].reshape(_NW, _NB, _BCH, _K)
    dst2 = edge_index[1].reshape(_NW, _EPW)
    zeros128 = jnp.zeros((_N, _D), jnp.float32)
    zerosn = jnp.zeros((_N,), jnp.float32)

    degp = _deg_sc(dst2, zerosn)
    y1, dinvf = _tc1(x, W1, degp)
    z1p = _edge_sc(y1, src, dst, zeros128)
    y2 = _tc2(z1p, y1, dinvf, b1.reshape(1, _D), W2)
    z2p = _edge_sc(y2, src, dst, zeros128)
    return _tc3(z2p, y2, dinvf, b2.reshape(1, _D), batch.reshape(1, _N),
                Wc.T, bc.reshape(1, _C))


# speedup vs baseline: 1.0343x; 1.0343x over previous
import functools

import jax
import jax.numpy as jnp
from jax import lax
from jax.experimental import pallas as pl
from jax.experimental.pallas import tpu as pltpu
from jax.experimental.pallas import tpu_sc as plsc

_N = 10000
_E = 320000
_D = 128
_G = 64
_C = 32

_NW = 32
_K = 80
_EPW = _E // _NW
_NCH = _EPW // _K
_DEPTH = 4
_GLAG = 3
_RPS = 624
_TAIL = _N - 16 * _RPS

_mesh = plsc.VectorSubcoreMesh(core_axis_name="c", subcore_axis_name="s")


def _sliced_copy(s, src, dst):
    pltpu.sync_copy(src.at[pl.ds(s * _RPS, _RPS)], dst.at[pl.ds(s * _RPS, _RPS)])

    @pl.when(s == 15)
    def _():
        pltpu.sync_copy(src.at[pl.ds(16 * _RPS, _TAIL)],
                        dst.at[pl.ds(16 * _RPS, _TAIL)])


@functools.partial(
    pl.kernel,
    out_type=jax.ShapeDtypeStruct((_NW, _N), jnp.float32),
    mesh=_mesh,
    compiler_params=pltpu.CompilerParams(needs_layout_passes=False),
    scratch_types=[
        pltpu.VMEM((_N,), jnp.int32),
        pltpu.VMEM((_N,), jnp.float32),
    ],
)
def _deg_sc(dst_hbm, zeros_hbm, out_hbm, idxv, hist):
    c = lax.axis_index("c")
    s = lax.axis_index("s")
    wid = s * 2 + c
    pltpu.sync_copy(dst_hbm.at[wid], idxv)
    pltpu.sync_copy(zeros_hbm, hist)
    ones = jnp.ones((16,), jnp.float32)

    def body(v, carry):
        idx = idxv[pl.ds(v * 16, 16)]
        plsc.addupdate_scatter(hist, [idx], ones)
        return carry

    lax.fori_loop(0, _EPW // 16, body, 0)
    pltpu.sync_copy(hist, out_hbm.at[wid])


_NB = 5
_BCH = _NCH // _NB


@functools.partial(
    pl.kernel,
    out_type=jax.ShapeDtypeStruct((2, _N, _D), jnp.float32),
    mesh=_mesh,
    scratch_types=[
        pltpu.VMEM((_BCH, _K), jnp.int32),
        pltpu.VMEM((_BCH, _K), jnp.int32),
        [pltpu.VMEM((_K, _D), jnp.float32)] * _DEPTH,
        pltpu.VMEM_SHARED((_N, _D), jnp.float32),
        [pltpu.SemaphoreType.DMA] * _DEPTH,
        [pltpu.SemaphoreType.DMA] * _DEPTH,
    ],
)
def _edge_sc(y_hbm, src_hbm, dst_hbm, zeros_hbm, out_hbm,
             srcv, dstv, rows, acc, gsem, ssem):
    c = lax.axis_index("c")
    s = lax.axis_index("s")
    wid = s * 2 + c
    _sliced_copy(s, zeros_hbm, acc)
    plsc.subcore_barrier()

    def gdesc(j, b):
        return pltpu.make_async_copy(y_hbm.at[srcv.at[j]], rows[b], gsem[b])

    def sdesc(j, b):
        return pltpu.make_async_copy(rows[b], acc.at[dstv.at[j]], ssem[b])

    def step(j, b):
        jmg = jnp.maximum(j - _GLAG, 0)
        jmd = jnp.maximum(j - _DEPTH, 0)

        @pl.when(jnp.logical_and(j >= _DEPTH, j <= _BCH + _DEPTH - 1))
        def _():
            sdesc(jmd, b).wait()

        @pl.when(j <= _BCH - 1)
        def _():
            gdesc(j, b).start()

        @pl.when(jnp.logical_and(j >= _GLAG, j <= _BCH + _GLAG - 1))
        def _():
            bb = (b - _GLAG) % _DEPTH
            gdesc(jmg, bb).wait()
            sdesc(jmg, bb).start(add=True)

    def body(u, carry):
        for t in range(_DEPTH):
            step(_DEPTH * u + t, t)
        return carry

    for blk in range(_NB):
        pltpu.sync_copy(src_hbm.at[wid, blk], srcv)
        pltpu.sync_copy(dst_hbm.at[wid, blk], dstv)
        lax.fori_loop(0, (_BCH + 2 * _DEPTH - 1) // _DEPTH, body, 0)

    plsc.subcore_barrier()
    _sliced_copy(s, acc, out_hbm.at[c])


def _tc1_body(x_ref, w1_ref, degp_ref, y1_ref, dinv_ref):
    deg_col = lax.dot_general(degp_ref[...], jnp.ones((_NW, 1), jnp.float32),
                              (((0,), (0,)), ((), ())),
                              preferred_element_type=jnp.float32)
    dinv = lax.rsqrt(deg_col + 1.0)
    xw = jnp.dot(x_ref[...], w1_ref[...], preferred_element_type=jnp.float32)
    y1_ref[...] = xw * dinv
    dinv_ref[...] = jnp.broadcast_to(dinv, (_N, _D))


_tc1 = pl.pallas_call(
    _tc1_body,
    out_shape=[
        jax.ShapeDtypeStruct((_N, _D), jnp.float32),
        jax.ShapeDtypeStruct((_N, _D), jnp.float32),
    ],
)


def _tc2_body(zp_ref, y1_ref, dinv_ref, b1_ref, w2_ref, y2_ref):
    z = zp_ref[0] + zp_ref[1] + y1_ref[...]
    h = jnp.maximum(z * dinv_ref[...] + b1_ref[...], 0.0)
    y2_ref[...] = jnp.dot(h, w2_ref[...],
                          preferred_element_type=jnp.float32) * dinv_ref[...]


_tc2 = pl.pallas_call(
    _tc2_body,
    out_shape=jax.ShapeDtypeStruct((_N, _D), jnp.float32),
)


def _tc3_body(zp_ref, y2_ref, dinv_ref, b2_ref, batch_ref, wct_ref, bc_ref,
              out_ref):
    z = zp_ref[0] + zp_ref[1] + y2_ref[...]
    h = jnp.maximum(z * dinv_ref[...] + b2_ref[...], 0.0)
    seg = lax.broadcasted_iota(jnp.int32, (_G, _N), 0)
    oht = (seg == batch_ref[...]).astype(jnp.float32)
    sums = jnp.dot(oht, h, preferred_element_type=jnp.float32)
    cnt = jnp.sum(oht, axis=1, keepdims=True)
    pooled = sums / jnp.maximum(cnt, 1.0)
    out_ref[...] = jnp.dot(pooled, wct_ref[...],
                           preferred_element_type=jnp.float32) + bc_ref[...]


_tc3 = pl.pallas_call(
    _tc3_body,
    out_shape=jax.ShapeDtypeStruct((_G, _C), jnp.float32),
)


def kernel(x, edge_index, batch, W1, b1, W2, b2, Wc, bc):
    src = edge_index[0].reshape(_NW, _NB, _BCH, _K)
    dst = edge_index[1].reshape(_NW, _NB, _BCH, _K)
    dst2 = edge_index[1].reshape(_NW, _EPW)
    zeros128 = jnp.zeros((_N, _D), jnp.float32)
    zerosn = jnp.zeros((_N,), jnp.float32)

    degp = _deg_sc(dst2, zerosn)
    y1, dinvf = _tc1(x, W1, degp)
    z1p = _edge_sc(y1, src, dst, zeros128)
    y2 = _tc2(z1p, y1, dinvf, b1.reshape(1, _D), W2)
    z2p = _edge_sc(y2, src, dst, zeros128)
    return _tc3(z2p, y2, dinvf, b2.reshape(1, _D), batch.reshape(1, _N),
                Wc.T, bc.reshape(1, _C))


# speedup vs baseline: 1.0554x; 1.0204x over previous
import functools

import jax
import jax.numpy as jnp
from jax import lax
from jax.experimental import pallas as pl
from jax.experimental.pallas import tpu as pltpu
from jax.experimental.pallas import tpu_sc as plsc

_N = 10000
_E = 320000
_D = 128
_G = 64
_C = 32

_NW = 32
_K = 80
_EPW = _E // _NW
_NCH = _EPW // _K
_DEPTH = 4
_GLAG = 3
_RPS = 624
_TAIL = _N - 16 * _RPS

_mesh = plsc.VectorSubcoreMesh(core_axis_name="c", subcore_axis_name="s")


def _sliced_copy(s, src, dst):
    pltpu.sync_copy(src.at[pl.ds(s * _RPS, _RPS)], dst.at[pl.ds(s * _RPS, _RPS)])

    @pl.when(s == 15)
    def _():
        pltpu.sync_copy(src.at[pl.ds(16 * _RPS, _TAIL)],
                        dst.at[pl.ds(16 * _RPS, _TAIL)])


@functools.partial(
    pl.kernel,
    out_type=jax.ShapeDtypeStruct((_NW, _N), jnp.float32),
    mesh=_mesh,
    compiler_params=pltpu.CompilerParams(needs_layout_passes=False),
    scratch_types=[
        pltpu.VMEM((_N,), jnp.int32),
        pltpu.VMEM((_N,), jnp.float32),
    ],
)
def _deg_sc(dst_hbm, zeros_hbm, out_hbm, idxv, hist):
    c = lax.axis_index("c")
    s = lax.axis_index("s")
    wid = s * 2 + c
    pltpu.sync_copy(dst_hbm.at[wid], idxv)
    pltpu.sync_copy(zeros_hbm, hist)
    ones = jnp.ones((16,), jnp.float32)

    def body(v, carry):
        idx = idxv[pl.ds(v * 16, 16)]
        plsc.addupdate_scatter(hist, [idx], ones)
        return carry

    lax.fori_loop(0, _EPW // 16, body, 0)
    pltpu.sync_copy(hist, out_hbm.at[wid])


_NB = 5
_BCH = _NCH // _NB


@functools.partial(
    pl.kernel,
    out_type=jax.ShapeDtypeStruct((2, _N, _D), jnp.float32),
    mesh=_mesh,
    scratch_types=[
        pltpu.VMEM((2, _BCH, _K), jnp.int32),
        [pltpu.VMEM((_K, _D), jnp.float32)] * _DEPTH,
        pltpu.VMEM_SHARED((_N, _D), jnp.float32),
        [pltpu.SemaphoreType.DMA] * _DEPTH,
        [pltpu.SemaphoreType.DMA] * _DEPTH,
    ],
)
def _edge_sc(y_hbm, idx_hbm, zeros_hbm, out_hbm,
             idxv, rows, acc, gsem, ssem):
    c = lax.axis_index("c")
    s = lax.axis_index("s")
    wid = s * 2 + c
    _sliced_copy(s, zeros_hbm, acc)
    plsc.subcore_barrier()

    def gdesc(j, b):
        return pltpu.make_async_copy(y_hbm.at[idxv.at[0, j]], rows[b], gsem[b])

    def sdesc(j, b):
        return pltpu.make_async_copy(rows[b], acc.at[idxv.at[1, j]], ssem[b])

    def step(j, b):
        jmg = jnp.maximum(j - _GLAG, 0)
        jmd = jnp.maximum(j - _DEPTH, 0)

        @pl.when(jnp.logical_and(j >= _DEPTH, j <= _BCH + _DEPTH - 1))
        def _():
            sdesc(jmd, b).wait()

        @pl.when(j <= _BCH - 1)
        def _():
            gdesc(j, b).start()

        @pl.when(jnp.logical_and(j >= _GLAG, j <= _BCH + _GLAG - 1))
        def _():
            bb = (b - _GLAG) % _DEPTH
            gdesc(jmg, bb).wait()
            sdesc(jmg, bb).start(add=True)

    def body(u, carry):
        for t in range(_DEPTH):
            step(_DEPTH * u + t, t)
        return carry

    for blk in range(_NB):
        pltpu.sync_copy(idx_hbm.at[wid, blk], idxv)
        lax.fori_loop(0, (_BCH + 2 * _DEPTH - 1) // _DEPTH, body, 0)

    plsc.subcore_barrier()
    _sliced_copy(s, acc, out_hbm.at[c])


def _tc1_body(x_ref, w1_ref, degp_ref, y1_ref, dinv_ref):
    deg_col = lax.dot_general(degp_ref[...], jnp.ones((_NW, 1), jnp.float32),
                              (((0,), (0,)), ((), ())),
                              preferred_element_type=jnp.float32)
    dinv = lax.rsqrt(deg_col + 1.0)
    xw = jnp.dot(x_ref[...], w1_ref[...], preferred_element_type=jnp.float32)
    y1_ref[...] = xw * dinv
    dinv_ref[...] = jnp.broadcast_to(dinv, (_N, _D))


_tc1 = pl.pallas_call(
    _tc1_body,
    out_shape=[
        jax.ShapeDtypeStruct((_N, _D), jnp.float32),
        jax.ShapeDtypeStruct((_N, _D), jnp.float32),
    ],
)


def _tc2_body(zp_ref, y1_ref, dinv_ref, b1_ref, w2_ref, y2_ref):
    z = zp_ref[0] + zp_ref[1] + y1_ref[...]
    h = jnp.maximum(z * dinv_ref[...] + b1_ref[...], 0.0)
    y2_ref[...] = jnp.dot(h, w2_ref[...],
                          preferred_element_type=jnp.float32) * dinv_ref[...]


_tc2 = pl.pallas_call(
    _tc2_body,
    out_shape=jax.ShapeDtypeStruct((_N, _D), jnp.float32),
)


def _tc3_body(zp_ref, y2_ref, dinv_ref, b2_ref, batch_ref, wct_ref, bc_ref,
              out_ref):
    z = zp_ref[0] + zp_ref[1] + y2_ref[...]
    h = jnp.maximum(z * dinv_ref[...] + b2_ref[...], 0.0)
    seg = lax.broadcasted_iota(jnp.int32, (_G, _N), 0)
    oht = (seg == batch_ref[...]).astype(jnp.float32)
    sums = jnp.dot(oht, h, preferred_element_type=jnp.float32)
    cnt = jnp.sum(oht, axis=1, keepdims=True)
    pooled = sums / jnp.maximum(cnt, 1.0)
    out_ref[...] = jnp.dot(pooled, wct_ref[...],
                           preferred_element_type=jnp.float32) + bc_ref[...]


_tc3 = pl.pallas_call(
    _tc3_body,
    out_shape=jax.ShapeDtypeStruct((_G, _C), jnp.float32),
)


def kernel(x, edge_index, batch, W1, b1, W2, b2, Wc, bc):
    idx4 = jnp.stack([edge_index[0].reshape(_NW, _NB, _BCH, _K),
                      edge_index[1].reshape(_NW, _NB, _BCH, _K)], axis=2)
    dst2 = edge_index[1].reshape(_NW, _EPW)
    zeros128 = jnp.zeros((_N, _D), jnp.float32)
    zerosn = jnp.zeros((_N,), jnp.float32)

    degp = _deg_sc(dst2, zerosn)
    y1, dinvf = _tc1(x, W1, degp)
    z1p = _edge_sc(y1, idx4, zeros128)
    y2 = _tc2(z1p, y1, dinvf, b1.reshape(1, _D), W2)
    z2p = _edge_sc(y2, idx4, zeros128)
    return _tc3(z2p, y2, dinvf, b2.reshape(1, _D), batch.reshape(1, _N),
                Wc.T, bc.reshape(1, _C))
